# Initial kernel scaffold; baseline (speedup 1.0000x reference)
import jax
import jax.numpy as jnp
from jax.experimental import pallas as pl


def kernel(encoder_cat, decoder_cat, enc_tables, dec_tables):
    raise NotImplementedError("write your pallas kernel here")


# speedup vs baseline: 5.7640x; 5.7640x over previous
import jax
import jax.numpy as jnp
from jax import lax
from jax.experimental import pallas as pl
from jax.experimental.pallas import tpu as pltpu
from jax.experimental.pallas import tpu_sc as plsc

B, S, F, V, D = 4096, 50, 4, 100000, 64
N = B * S
N4 = N * F
NC, NS = 2, 16
NW = NC * NS
ROWS_PER_W = N // NW
CHUNK = 256
IDX_PER_CHUNK = CHUNK * F
G = IDX_PER_CHUNK // 128
CHUNKS = ROWS_PER_W // CHUNK
IDX_ROWS_PER_W = ROWS_PER_W * F // 128


def _body(enc_idx, dec_idx, enc_tab, dec_tab, enc_out, dec_out,
          idx_v, rows_v, acc_v, sem):
    wid = lax.axis_index("s") * NC + lax.axis_index("c")
    offv = (lax.iota(jnp.int32, 16) % 4) * V

    for idx_hbm, tab_hbm, out_hbm in ((enc_idx, enc_tab, enc_out),
                                      (dec_idx, dec_tab, dec_out)):
        def chunk_body(ch, _, idx_hbm=idx_hbm, tab_hbm=tab_hbm, out_hbm=out_hbm):
            idx_row = wid * IDX_ROWS_PER_W + ch * G
            out_base = wid * ROWS_PER_W + ch * CHUNK
            pltpu.sync_copy(idx_hbm.at[pl.ds(idx_row, G)], idx_v)
            for j in range(G):
                for i in range(8):
                    sl = (j, pl.ds(i * 16, 16))
                    idx_v[sl] = idx_v[sl] + offv
            cps = [
                pltpu.async_copy(tab_hbm.at[idx_v.at[j]],
                                 rows_v.at[pl.ds(j * 128, 128)], sem)
                for j in range(G)
            ]
            for cp in cps:
                cp.wait()

            def row_body(c, _):
                r = c * F
                for k in range(D // 16):
                    sl = pl.ds(k * 16, 16)
                    v = rows_v[r, sl]
                    v = v + rows_v[r + 1, sl]
                    v = v + rows_v[r + 2, sl]
                    v = v + rows_v[r + 3, sl]
                    acc_v[c, sl] = v
                return 0

            lax.fori_loop(0, CHUNK, row_body, 0, unroll=2)
            pltpu.sync_copy(acc_v, out_hbm.at[pl.ds(out_base, CHUNK)])
            return 0

        lax.fori_loop(0, CHUNKS, chunk_body, 0)


def kernel(encoder_cat, decoder_cat, enc_tables, dec_tables):
    enc_idx = encoder_cat.astype(jnp.int32).reshape(N4 // 128, 128)
    dec_idx = decoder_cat.astype(jnp.int32).reshape(N4 // 128, 128)
    enc_tab = enc_tables.reshape(F * V, D)
    dec_tab = dec_tables.reshape(F * V, D)

    mesh = plsc.VectorSubcoreMesh(core_axis_name="c", subcore_axis_name="s")
    k = pl.kernel(
        _body,
        out_type=(
            jax.ShapeDtypeStruct((N, D), jnp.float32),
            jax.ShapeDtypeStruct((N, D), jnp.float32),
        ),
        mesh=mesh,
        compiler_params=pltpu.CompilerParams(use_tc_tiling_on_sc=False),
        scratch_types=[
            pltpu.VMEM((G, 128), jnp.int32),
            pltpu.VMEM((IDX_PER_CHUNK, D), jnp.float32),
            pltpu.VMEM((CHUNK, D), jnp.float32),
            pltpu.SemaphoreType.DMA,
        ],
    )
    enc_out, dec_out = k(enc_idx, dec_idx, enc_tab, dec_tab)
    return enc_out.reshape(B, S, D), dec_out.reshape(B, S, D)


# speedup vs baseline: 8.3960x; 1.4566x over previous
import jax
import jax.numpy as jnp
from jax import lax
from jax.experimental import pallas as pl
from jax.experimental.pallas import tpu as pltpu
from jax.experimental.pallas import tpu_sc as plsc

B, S, F, V, D = 4096, 50, 4, 100000, 64
N = B * S
NC, NS = 2, 16
NW = NC * NS
ROWS_PER_W = N // NW
CHUNK = 256
G = CHUNK // 128
CHUNKS = ROWS_PER_W // CHUNK
IDX_ROWS = N // 128


def _body(enc_idx, dec_idx, enc_tab, dec_tab, enc_out, dec_out,
          idx_v, acc_v, sem):
    wid = lax.axis_index("s") * NC + lax.axis_index("c")

    for idx_hbm, tab_hbm, out_hbm in ((enc_idx, enc_tab, enc_out),
                                      (dec_idx, dec_tab, dec_out)):
        def chunk_body(ch, _, idx_hbm=idx_hbm, tab_hbm=tab_hbm, out_hbm=out_hbm):
            idx_row = wid * (ROWS_PER_W // 128) + ch * G
            out_base = wid * ROWS_PER_W + ch * CHUNK
            for f in range(F):
                pltpu.sync_copy(idx_hbm.at[f].at[pl.ds(idx_row, G)],
                                idx_v.at[f])
            for f in range(1, F):
                off = jnp.full((16,), f * V, jnp.int32)
                for i in range(CHUNK // 16):
                    sl = (f, i // 8, pl.ds((i % 8) * 16, 16))
                    idx_v[sl] = idx_v[sl] + off
            cps = [
                pltpu.async_copy(tab_hbm.at[idx_v.at[0].at[j]],
                                 acc_v.at[pl.ds(j * 128, 128)], sem)
                for j in range(G)
            ]
            for cp in cps:
                cp.wait()
            cps = [
                pltpu.async_copy(tab_hbm.at[idx_v.at[f].at[j]],
                                 acc_v.at[pl.ds(j * 128, 128)], sem, add=True)
                for f in range(1, F)
                for j in range(G)
            ]
            for cp in cps:
                cp.wait()
            pltpu.sync_copy(acc_v, out_hbm.at[pl.ds(out_base, CHUNK)])
            return 0

        lax.fori_loop(0, CHUNKS, chunk_body, 0)


def kernel(encoder_cat, decoder_cat, enc_tables, dec_tables):
    enc_idx = (encoder_cat.astype(jnp.int32).reshape(N, F).T
               .reshape(F, IDX_ROWS, 128))
    dec_idx = (decoder_cat.astype(jnp.int32).reshape(N, F).T
               .reshape(F, IDX_ROWS, 128))
    enc_tab = enc_tables.reshape(F * V, D)
    dec_tab = dec_tables.reshape(F * V, D)

    mesh = plsc.VectorSubcoreMesh(core_axis_name="c", subcore_axis_name="s")
    k = pl.kernel(
        _body,
        out_type=(
            jax.ShapeDtypeStruct((N, D), jnp.float32),
            jax.ShapeDtypeStruct((N, D), jnp.float32),
        ),
        mesh=mesh,
        compiler_params=pltpu.CompilerParams(use_tc_tiling_on_sc=False),
        scratch_types=[
            pltpu.VMEM((F, G, 128), jnp.int32),
            pltpu.VMEM((CHUNK, D), jnp.float32),
            pltpu.SemaphoreType.DMA,
        ],
    )
    enc_out, dec_out = k(enc_idx, dec_idx, enc_tab, dec_tab)
    return enc_out.reshape(B, S, D), dec_out.reshape(B, S, D)


# speedup vs baseline: 9.6184x; 1.1456x over previous
import jax
import jax.numpy as jnp
from jax import lax
from jax.experimental import pallas as pl
from jax.experimental.pallas import tpu as pltpu
from jax.experimental.pallas import tpu_sc as plsc

B, S, F, V, D = 4096, 50, 4, 100000, 64
N = B * S
NC, NS = 2, 16
NW = NC * NS
ROWS_PER_W = N // NW
CHUNK = 128
CHUNKS = ROWS_PER_W // CHUNK
NBLK = N // CHUNK


def _body(enc_idx, dec_idx, enc_tab, dec_tab, enc_out, dec_out,
          idx_v, acc_v, sem_idx, sem_g0, sem_add, sem_out):
    wid = lax.axis_index("s") * NC + lax.axis_index("c")
    blk0 = wid * CHUNKS
    out0 = wid * ROWS_PER_W

    for side, (idx_hbm, tab_hbm, out_hbm) in enumerate(
            ((enc_idx, enc_tab, enc_out), (dec_idx, dec_tab, dec_out))):

        def fire_idx(t, b):
            pltpu.async_copy(idx_hbm.at[blk0 + t], idx_v.at[b],
                             sem_idx.at[b])

        def wait_idx_and_offset(b):
            pltpu.make_async_copy(idx_hbm.at[blk0], idx_v.at[b],
                                  sem_idx.at[b]).wait()
            for f in range(1, F):
                off = jnp.full((16,), f * V, jnp.int32)
                for i in range(CHUNK // 16):
                    sl = (b, f, pl.ds(i * 16, 16))
                    idx_v[sl] = idx_v[sl] + off

        def fire_g0(b):
            pltpu.async_copy(tab_hbm.at[idx_v.at[b, 0]], acc_v.at[b],
                             sem_g0.at[b])

        def wait_g0(b):
            pltpu.make_async_copy(tab_hbm.at[idx_v.at[b, 0]], acc_v.at[b],
                                  sem_g0.at[b]).wait()

        def fire_adds(t, b):
            for f in range(1, F):
                pltpu.async_copy(tab_hbm.at[idx_v.at[b, f]], acc_v.at[b],
                                 sem_add.at[b], add=True)

        def wait_adds(b):
            for f in range(1, F):
                pltpu.make_async_copy(tab_hbm.at[idx_v.at[b, f]],
                                      acc_v.at[b], sem_add.at[b]).wait()

        def fire_out(t, b):
            pltpu.async_copy(acc_v.at[b],
                             out_hbm.at[pl.ds(out0 + t * CHUNK, CHUNK)],
                             sem_out.at[b])

        def wait_out(b):
            pltpu.make_async_copy(acc_v.at[b],
                                  out_hbm.at[pl.ds(out0, CHUNK)],
                                  sem_out.at[b]).wait()

        def steady(t, b, first=False):
            nb = 1 - b
            fire_idx(t + 1, nb)
            wait_g0(b)
            fire_adds(t, b)
            wait_idx_and_offset(nb)
            if not first:
                wait_out(nb)
            fire_g0(nb)
            wait_adds(b)
            fire_out(t, b)

        if side == 1:
            wait_out(0)
        fire_idx(0, 0)
        wait_idx_and_offset(0)
        fire_g0(0)
        steady(0, 0, first=(side == 0))

        def pair(p, _):
            t = 1 + 2 * p
            steady(t, 1)
            steady(t + 1, 0)
            return 0

        lax.fori_loop(0, (CHUNKS - 2) // 2, pair, 0)

        b = 1
        wait_g0(b)
        fire_adds(CHUNKS - 1, b)
        wait_adds(b)
        fire_out(CHUNKS - 1, b)

    wait_out(0)
    wait_out(1)


def kernel(encoder_cat, decoder_cat, enc_tables, dec_tables):
    enc_idx = (encoder_cat.astype(jnp.int32).reshape(NBLK, CHUNK, F)
               .transpose(0, 2, 1))
    dec_idx = (decoder_cat.astype(jnp.int32).reshape(NBLK, CHUNK, F)
               .transpose(0, 2, 1))
    enc_tab = enc_tables.reshape(F * V, D)
    dec_tab = dec_tables.reshape(F * V, D)

    mesh = plsc.VectorSubcoreMesh(core_axis_name="c", subcore_axis_name="s")
    k = pl.kernel(
        _body,
        out_type=(
            jax.ShapeDtypeStruct((N, D), jnp.float32),
            jax.ShapeDtypeStruct((N, D), jnp.float32),
        ),
        mesh=mesh,
        compiler_params=pltpu.CompilerParams(use_tc_tiling_on_sc=False),
        scratch_types=[
            pltpu.VMEM((2, F, CHUNK), jnp.int32),
            pltpu.VMEM((2, CHUNK, D), jnp.float32),
            pltpu.SemaphoreType.DMA((2,)),
            pltpu.SemaphoreType.DMA((2,)),
            pltpu.SemaphoreType.DMA((2,)),
            pltpu.SemaphoreType.DMA((2,)),
        ],
    )
    enc_out, dec_out = k(enc_idx, dec_idx, enc_tab, dec_tab)
    return enc_out.reshape(B, S, D), dec_out.reshape(B, S, D)


# speedup vs baseline: 12.8440x; 1.3354x over previous
import jax
import jax.numpy as jnp
from jax import lax
from jax.experimental import pallas as pl
from jax.experimental.pallas import tpu as pltpu
from jax.experimental.pallas import tpu_sc as plsc

B, S, F, V, D = 4096, 50, 4, 100000, 64
N = B * S
NC, NS = 2, 16
NW = NC * NS
BH = B // 128
TILES = S * BH
TPW = TILES // NW


def _body(enc_idx, dec_idx, enc_tab, dec_tab, enc_out, dec_out,
          idx_v, acc_v, outt_v, sem_idx, sem_g, sem_out):
    wid = lax.axis_index("s") * NC + lax.axis_index("c")
    t0 = wid * TPW
    iota = lax.iota(jnp.int32, 16)
    dhvecs = [(iota + 16 * k) // 8 for k in range(4)]
    dlvecs = [(iota + 16 * k) % 8 for k in range(4)]
    zeros = jnp.zeros((16,), jnp.float32)

    def z_body(r, _):
        for b in range(2):
            for k in range(4):
                acc_v[b, r, pl.ds(16 * k, 16)] = zeros
        return 0
    lax.fori_loop(0, 128, z_body, 0)

    def tsl(t):
        s = (t0 + t) // BH
        bh = (t0 + t) % BH
        return s, bh

    for side, (idx_hbm, tab_hbm, out_hbm) in enumerate(
            ((enc_idx, enc_tab, enc_out), (dec_idx, dec_tab, dec_out))):
        def fire_idx(t, b):
            s, bh = tsl(t)
            pltpu.async_copy(idx_hbm.at[s, bh], idx_v.at[b], sem_idx.at[b])

        def wait_idx_off(b):
            pltpu.make_async_copy(idx_hbm.at[0, 0], idx_v.at[b],
                                  sem_idx.at[b]).wait()
            for f in range(1, F):
                off = jnp.full((16,), f * V, jnp.int32)
                for i in range(128 // 16):
                    sl = (b, f, pl.ds(i * 16, 16))
                    idx_v[sl] = idx_v[sl] + off

        def fire_gathers(b):
            for f in range(F):
                pltpu.async_copy(tab_hbm.at[idx_v.at[b, f]], acc_v.at[b],
                                 sem_g.at[b], add=True)

        def wait_gathers(b):
            for f in range(F):
                pltpu.make_async_copy(tab_hbm.at[idx_v.at[b, f]],
                                      acc_v.at[b], sem_g.at[b]).wait()

        def trans_zero(b):
            def b_body(r, _):
                bvec = jnp.full((16,), 0, jnp.int32) + r
                for k in range(4):
                    v = acc_v[b, r, pl.ds(16 * k, 16)]
                    plsc.store_scatter(outt_v.at[b],
                                       [dhvecs[k], dlvecs[k], bvec], v)
                    acc_v[b, r, pl.ds(16 * k, 16)] = zeros
                return 0
            lax.fori_loop(0, 128, b_body, 0)

        def fire_out(t, b):
            s, bh = tsl(t)
            pltpu.async_copy(outt_v.at[b, :, :, pl.ds(0, 128)],
                             out_hbm.at[s, :, bh], sem_out.at[b])

        def wait_out(b):
            pltpu.make_async_copy(outt_v.at[b, :, :, pl.ds(0, 128)],
                                  out_hbm.at[0, :, 0],
                                  sem_out.at[b]).wait()

        def iter_t(t, b, do_waitout=True, do_trans=True, do_next=True):
            nb = 1 - b
            if do_next:
                fire_idx(t + 1, nb)
            if do_waitout:
                wait_out(nb)
            if do_trans:
                trans_zero(nb)
                fire_out(t - 1, nb)
            if do_next:
                wait_idx_off(nb)
            wait_gathers(b)
            if do_next:
                fire_gathers(nb)

        fire_idx(0, 0)
        wait_idx_off(0)
        fire_gathers(0)
        iter_t(0, 0, do_waitout=False, do_trans=False)
        iter_t(1, 1, do_waitout=(side == 1))
        iter_t(2, 0, do_waitout=(side == 1))

        def pair(p, _):
            t = 3 + 2 * p
            iter_t(t, 1)
            iter_t(t + 1, 0)
            return 0

        lax.fori_loop(0, (TPW - 4) // 2, pair, 0)

        iter_t(TPW - 1, 1, do_next=False)
        wait_out(1)
        trans_zero(1)
        fire_out(TPW - 1, 1)

    wait_out(0)
    wait_out(1)


def kernel(encoder_cat, decoder_cat, enc_tables, dec_tables):
    def to_idx5(cat):
        return (cat.astype(jnp.int32).transpose(1, 2, 0)
                .reshape(S, F, BH, 128).transpose(0, 2, 1, 3))

    enc_idx = to_idx5(encoder_cat)
    dec_idx = to_idx5(decoder_cat)
    enc_tab = enc_tables.reshape(F * V, D)
    dec_tab = dec_tables.reshape(F * V, D)

    mesh = plsc.VectorSubcoreMesh(core_axis_name="c", subcore_axis_name="s")
    k = pl.kernel(
        _body,
        out_type=(
            jax.ShapeDtypeStruct((S, 8, BH, 8, 128), jnp.float32),
            jax.ShapeDtypeStruct((S, 8, BH, 8, 128), jnp.float32),
        ),
        mesh=mesh,
        compiler_params=pltpu.CompilerParams(use_tc_tiling_on_sc=False,
                                             needs_layout_passes=False),
        scratch_types=[
            pltpu.VMEM((2, F, 128), jnp.int32),
            pltpu.VMEM((2, 128, D), jnp.float32),
            pltpu.VMEM((2, 8, 8, 129), jnp.float32),
            pltpu.SemaphoreType.DMA((2,)),
            pltpu.SemaphoreType.DMA((2,)),
            pltpu.SemaphoreType.DMA((2,)),
        ],
    )
    enc5, dec5 = k(enc_idx, dec_idx, enc_tab, dec_tab)

    def from_out5(y):
        return y.transpose(2, 4, 0, 1, 3).reshape(B, S, D)

    return from_out5(enc5), from_out5(dec5)


# speedup vs baseline: 13.7807x; 1.0729x over previous
import jax
import jax.numpy as jnp
from jax import lax
from jax.experimental import pallas as pl
from jax.experimental.pallas import tpu as pltpu
from jax.experimental.pallas import tpu_sc as plsc

B, S, F, V, D = 4096, 50, 4, 100000, 64
N = B * S
NC, NS = 2, 16
NW = NC * NS
BH = B // 128
TILES = S * BH
TPW = TILES // NW


def _body(idx_hbm, tab_hbm, out_hbm, idx_v, acc_v, outt_v,
          sem_idx, sem_g, sem_out):
    wid = lax.axis_index("s") * NC + lax.axis_index("c")
    t0 = wid * TPW
    iota = lax.iota(jnp.int32, 16)
    dhvecs = [(iota + 16 * k) // 8 for k in range(4)]
    dlvecs = [(iota + 16 * k) % 8 for k in range(4)]
    zeros = jnp.zeros((16,), jnp.float32)

    def z_body(r, _):
        for b in range(2):
            for k in range(4):
                acc_v[b, r, pl.ds(16 * k, 16)] = zeros
        return 0
    lax.fori_loop(0, 128, z_body, 0)

    def tsl(t):
        s = (t0 + t) // BH
        bh = (t0 + t) % BH
        return s, bh

    def fire_idx(t, b):
        s, bh = tsl(t)
        pltpu.async_copy(idx_hbm.at[s, bh], idx_v.at[b], sem_idx.at[b])

    def wait_idx_off(b):
        pltpu.make_async_copy(idx_hbm.at[0, 0], idx_v.at[b],
                              sem_idx.at[b]).wait()
        for f in range(F):
            off = jnp.full((16,), 2 * f * V, jnp.int32)
            for i in range(128 // 16):
                sl = (b, f, pl.ds(i * 16, 16))
                idx_v[sl] = idx_v[sl] * 2 + off

    def fire_gathers(b):
        for f in range(F):
            pltpu.async_copy(tab_hbm.at[idx_v.at[b, f]], acc_v.at[b],
                             sem_g.at[b], add=True)

    def wait_gathers(b):
        for f in range(F):
            pltpu.make_async_copy(tab_hbm.at[idx_v.at[b, f]],
                                  acc_v.at[b], sem_g.at[b]).wait()

    def trans_zero(b):
        def b_body(r, _):
            bvec = jnp.full((16,), 0, jnp.int32) + r
            for k in range(4):
                v = acc_v[b, r, pl.ds(16 * k, 16)]
                plsc.store_scatter(outt_v.at[b],
                                   [dhvecs[k], dlvecs[k], bvec], v)
                acc_v[b, r, pl.ds(16 * k, 16)] = zeros
            return 0
        lax.fori_loop(0, 128, b_body, 0)

    def fire_out(t, b):
        s, bh = tsl(t)
        pltpu.async_copy(outt_v.at[b, :, :, pl.ds(0, 128)],
                         out_hbm.at[s, :, bh], sem_out.at[b])

    def wait_out(b):
        pltpu.make_async_copy(outt_v.at[b, :, :, pl.ds(0, 128)],
                              out_hbm.at[0, :, 0],
                              sem_out.at[b]).wait()

    def iter_t(t, b, do_waitout=True, do_trans=True, do_next=True):
        nb = 1 - b
        if do_next:
            fire_idx(t + 1, nb)
        if do_waitout:
            wait_out(nb)
        if do_trans:
            trans_zero(nb)
            fire_out(t - 1, nb)
        if do_next:
            wait_idx_off(nb)
        wait_gathers(b)
        if do_next:
            fire_gathers(nb)

    fire_idx(0, 0)
    wait_idx_off(0)
    fire_gathers(0)
    iter_t(0, 0, do_waitout=False, do_trans=False)
    iter_t(1, 1, do_waitout=False)
    iter_t(2, 0, do_waitout=False)

    def pair(p, _):
        t = 3 + 2 * p
        iter_t(t, 1)
        iter_t(t + 1, 0)
        return 0

    lax.fori_loop(0, (TPW - 4) // 2, pair, 0)

    iter_t(TPW - 1, 1, do_next=False)
    wait_out(1)
    trans_zero(1)
    fire_out(TPW - 1, 1)
    wait_out(0)
    wait_out(1)


def kernel(encoder_cat, decoder_cat, enc_tables, dec_tables):
    def to_idx5(cat):
        return (cat.astype(jnp.int32).transpose(1, 2, 0)
                .reshape(S, F, BH, 128).transpose(0, 2, 1, 3))

    def to_tab(t):
        return jnp.pad(t, ((0, 0), (0, 0), (0, D))).reshape(2 * F * V, D)

    mesh = plsc.VectorSubcoreMesh(core_axis_name="c", subcore_axis_name="s")
    k = pl.kernel(
        _body,
        out_type=jax.ShapeDtypeStruct((S, 8, BH, 8, 128), jnp.float32),
        mesh=mesh,
        compiler_params=pltpu.CompilerParams(use_tc_tiling_on_sc=False,
                                             needs_layout_passes=False),
        scratch_types=[
            pltpu.VMEM((2, F, 128), jnp.int32),
            pltpu.VMEM((2, 128, D), jnp.float32),
            pltpu.VMEM((2, 8, 8, 129), jnp.float32),
            pltpu.SemaphoreType.DMA((2,)),
            pltpu.SemaphoreType.DMA((2,)),
            pltpu.SemaphoreType.DMA((2,)),
        ],
    )
    enc5 = k(to_idx5(encoder_cat), to_tab(enc_tables))
    dec5 = k(to_idx5(decoder_cat), to_tab(dec_tables))

    def from_out5(y):
        return y.transpose(2, 4, 0, 1, 3).reshape(B, S, D)

    return from_out5(enc5), from_out5(dec5)
